# Initial kernel scaffold; baseline (speedup 1.0000x reference)
import functools

import jax
import jax.numpy as jnp
from jax import lax
from jax.experimental import pallas as pl
from jax.experimental.pallas import tpu as pltpu
from jax.experimental.pallas import tpu_sc as plsc

_NC = 2
_NS = 16
_R = 125
_RP = 128


@functools.lru_cache(maxsize=None)
def _build_sc_call(n, d, b, nblk_w):
    mesh = plsc.VectorSubcoreMesh(core_axis_name="c", subcore_axis_name="s")

    @functools.partial(
        pl.kernel,
        out_type=(
            jax.ShapeDtypeStruct((_NC, b, d), jnp.float32),
            jax.ShapeDtypeStruct((_NC, b, 16), jnp.float32),
        ),
        mesh=mesh,
        scratch_types=[
            pltpu.VMEM((_RP, d), jnp.float32),
            pltpu.VMEM((nblk_w, _RP), jnp.int32),
            pltpu.VMEM((_RP, 16), jnp.float32),
            pltpu.VMEM((b, d), jnp.float32),
            pltpu.VMEM((b, 16), jnp.float32),
            pltpu.VMEM_SHARED((b, d), jnp.float32),
            pltpu.VMEM_SHARED((b, 16), jnp.float32),
        ],
    )
    def sc_call(feats_hbm, ids_hbm, psum_hbm, pcnt_hbm,
                rows_v, ids_v, ones_v, zf_v, zc_v, acc_sh, cnt_sh):
        c = lax.axis_index("c")
        s = lax.axis_index("s")
        wid = s * _NC + c

        zeros16 = jnp.zeros((16,), jnp.float32)
        ones16 = jnp.ones((16,), jnp.float32)

        def fill_ones(r, carry):
            ones_v[r, :] = ones16
            return carry

        def fill_ones_pad(r, carry):
            ones_v[r, :] = zeros16
            for cc in range(d // 16):
                rows_v[r, pl.ds(cc * 16, 16)] = zeros16
            return carry

        def fill_zf(r, carry):
            for cc in range(d // 16):
                zf_v[r, pl.ds(cc * 16, 16)] = zeros16
            zc_v[r, :] = zeros16
            return carry

        lax.fori_loop(0, _R, fill_ones, 0)
        lax.fori_loop(_R, _RP, fill_ones_pad, 0)
        lax.fori_loop(0, b, fill_zf, 0)

        @pl.when(s == 0)
        def _():
            pltpu.sync_copy(zf_v, acc_sh)
            pltpu.sync_copy(zc_v, cnt_sh)

        plsc.subcore_barrier()

        pltpu.sync_copy(ids_hbm.at[pl.ds(wid * nblk_w, nblk_w)], ids_v)

        def body(kk, carry):
            row0 = (wid * nblk_w + kk) * _R
            pltpu.sync_copy(feats_hbm.at[pl.ds(row0, _R)],
                            rows_v.at[pl.ds(0, _R)])
            idx = ids_v.at[kk]
            pltpu.sync_copy(rows_v, acc_sh.at[idx], add=True)
            pltpu.sync_copy(ones_v, cnt_sh.at[idx], add=True)
            return carry

        lax.fori_loop(0, nblk_w, body, 0)

        plsc.subcore_barrier()

        @pl.when(s == 0)
        def _():
            pltpu.sync_copy(acc_sh, psum_hbm.at[c])
            pltpu.sync_copy(cnt_sh, pcnt_hbm.at[c])

    return sc_call


def kernel(sfb, node_feats, segment_ids):
    n, d = node_feats.shape
    b = sfb.shape[0]
    nblk = n // _R
    nblk_w = nblk // (_NC * _NS)
    assert nblk * _R == n and nblk_w * _NC * _NS == nblk and d % 16 == 0

    ids = segment_ids.astype(jnp.int32).reshape(nblk, _R)
    ids_pad = jnp.pad(ids, ((0, 0), (0, _RP - _R)))

    psum, pcnt = _build_sc_call(n, d, b, nblk_w)(node_feats, ids_pad)

    sums = psum[0] + psum[1]
    cnt = pcnt[0, :, 0] + pcnt[1, :, 0]
    g_feat = sums / jnp.maximum(cnt, 1.0)[:, None]
    return jnp.concatenate(
        (sfb, g_feat.reshape(b, d, 1, 1, 1)), axis=1)


# speedup vs baseline: 5.0729x; 5.0729x over previous
import functools

import jax
import jax.numpy as jnp
from jax import lax
from jax.experimental import pallas as pl
from jax.experimental.pallas import tpu as pltpu
from jax.experimental.pallas import tpu_sc as plsc

_NC = 2
_NS = 16
_R = 125
_RP = 128


@functools.lru_cache(maxsize=None)
def _build_sc_call(n, d, b, nblk_w):
    mesh = plsc.VectorSubcoreMesh(core_axis_name="c", subcore_axis_name="s")

    @functools.partial(
        pl.kernel,
        out_type=(
            jax.ShapeDtypeStruct((_NC, b, d), jnp.float32),
            jax.ShapeDtypeStruct((_NC, b, 16), jnp.float32),
        ),
        mesh=mesh,
        compiler_params=pltpu.CompilerParams(use_tc_tiling_on_sc=False),
        scratch_types=[
            pltpu.VMEM((_RP, d), jnp.float32),
            pltpu.VMEM((nblk_w, _RP), jnp.int32),
            pltpu.VMEM((_RP, 16), jnp.float32),
            pltpu.VMEM((b, d), jnp.float32),
            pltpu.VMEM((b, 16), jnp.float32),
            pltpu.VMEM_SHARED((b, d), jnp.float32),
            pltpu.VMEM_SHARED((b, 16), jnp.float32),
        ],
    )
    def sc_call(feats_hbm, ids_hbm, psum_hbm, pcnt_hbm,
                rows_v, ids_v, ones_v, zf_v, zc_v, acc_sh, cnt_sh):
        c = lax.axis_index("c")
        s = lax.axis_index("s")
        wid = s * _NC + c

        zeros16 = jnp.zeros((16,), jnp.float32)
        ones16 = jnp.ones((16,), jnp.float32)

        def fill_ones(r, carry):
            ones_v[r, :] = ones16
            return carry

        def fill_ones_pad(r, carry):
            ones_v[r, :] = zeros16
            for cc in range(d // 16):
                rows_v[r, pl.ds(cc * 16, 16)] = zeros16
            return carry

        def fill_zf(r, carry):
            for cc in range(d // 16):
                zf_v[r, pl.ds(cc * 16, 16)] = zeros16
            zc_v[r, :] = zeros16
            return carry

        lax.fori_loop(0, _R, fill_ones, 0)
        lax.fori_loop(_R, _RP, fill_ones_pad, 0)
        lax.fori_loop(0, b, fill_zf, 0)

        @pl.when(s == 0)
        def _():
            pltpu.sync_copy(zf_v, acc_sh)
            pltpu.sync_copy(zc_v, cnt_sh)

        plsc.subcore_barrier()

        pltpu.sync_copy(ids_hbm.at[pl.ds(wid * nblk_w, nblk_w)], ids_v)

        def body(kk, carry):
            row0 = (wid * nblk_w + kk) * _R
            pltpu.sync_copy(feats_hbm.at[pl.ds(row0, _R)],
                            rows_v.at[pl.ds(0, _R)])
            idx = ids_v.at[kk]
            pltpu.sync_copy(rows_v, acc_sh.at[idx], add=True)
            pltpu.sync_copy(ones_v, cnt_sh.at[idx], add=True)
            return carry

        lax.fori_loop(0, nblk_w, body, 0)

        plsc.subcore_barrier()

        @pl.when(s == 0)
        def _():
            pltpu.sync_copy(acc_sh, psum_hbm.at[c])
            pltpu.sync_copy(cnt_sh, pcnt_hbm.at[c])

    return sc_call


def kernel(sfb, node_feats, segment_ids):
    n, d = node_feats.shape
    b = sfb.shape[0]
    nblk = n // _R
    nblk_w = nblk // (_NC * _NS)
    assert nblk * _R == n and nblk_w * _NC * _NS == nblk and d % 16 == 0

    ids = segment_ids.astype(jnp.int32).reshape(nblk, _R)
    ids_pad = jnp.pad(ids, ((0, 0), (0, _RP - _R)))

    psum, pcnt = _build_sc_call(n, d, b, nblk_w)(node_feats, ids_pad)

    sums = psum[0] + psum[1]
    cnt = pcnt[0, :, 0] + pcnt[1, :, 0]
    g_feat = sums / jnp.maximum(cnt, 1.0)[:, None]
    return jnp.concatenate(
        (sfb, g_feat.reshape(b, d, 1, 1, 1)), axis=1)


# speedup vs baseline: 5.8009x; 1.1435x over previous
import functools

import jax
import jax.numpy as jnp
from jax import lax
from jax.experimental import pallas as pl
from jax.experimental.pallas import tpu as pltpu
from jax.experimental.pallas import tpu_sc as plsc

_NC = 2
_NS = 16
_R = 125
_RP = 128


@functools.lru_cache(maxsize=None)
def _build_sc_call(n, d, b, nblk_w):
    mesh = plsc.VectorSubcoreMesh(core_axis_name="c", subcore_axis_name="s")

    @functools.partial(
        pl.kernel,
        out_type=(
            jax.ShapeDtypeStruct((_NC, b, d), jnp.float32),
            jax.ShapeDtypeStruct((_NC, b, 16), jnp.float32),
        ),
        mesh=mesh,
        compiler_params=pltpu.CompilerParams(use_tc_tiling_on_sc=False),
        scratch_types=[
            pltpu.VMEM((2, _RP, d), jnp.float32),
            pltpu.VMEM((nblk_w, _RP), jnp.int32),
            pltpu.VMEM((_RP, 16), jnp.float32),
            pltpu.VMEM((b, d), jnp.float32),
            pltpu.VMEM((b, 16), jnp.float32),
            pltpu.SemaphoreType.DMA,
            pltpu.VMEM_SHARED((b, d), jnp.float32),
            pltpu.VMEM_SHARED((b, 16), jnp.float32),
        ],
    )
    def sc_call(feats_hbm, ids_hbm, psum_hbm, pcnt_hbm,
                rows_v, ids_v, ones_v, zf_v, zc_v, sem,
                acc_sh, cnt_sh):
        c = lax.axis_index("c")
        s = lax.axis_index("s")
        wid = s * _NC + c

        zeros16 = jnp.zeros((16,), jnp.float32)
        ones16 = jnp.ones((16,), jnp.float32)
        def fill_ones(r, carry):
            ones_v[r, :] = ones16
            return carry

        def fill_ones_pad(r, carry):
            ones_v[r, :] = zeros16
            for bb in range(2):
                for cc in range(d // 16):
                    rows_v[bb, r, pl.ds(cc * 16, 16)] = zeros16
            return carry

        def fill_zf(r, carry):
            for cc in range(d // 16):
                zf_v[r, pl.ds(cc * 16, 16)] = zeros16
            zc_v[r, :] = zeros16
            return carry

        lax.fori_loop(0, _R, fill_ones, 0)
        lax.fori_loop(_R, _RP, fill_ones_pad, 0)
        lax.fori_loop(0, b, fill_zf, 0)

        @pl.when(s == 0)
        def _():
            pltpu.sync_copy(zf_v, acc_sh)
            pltpu.sync_copy(zc_v, cnt_sh)

        plsc.subcore_barrier()

        pltpu.sync_copy(ids_hbm.at[pl.ds(wid * nblk_w, nblk_w)], ids_v)

        def row_dma(kk, buf):
            row0 = (wid * nblk_w + kk) * _R
            return pltpu.async_copy(feats_hbm.at[pl.ds(row0, _R)],
                                    rows_v.at[buf, pl.ds(0, _R)], sem)

        row_dma(0, 0)

        def pair_body(g, carry):
            for b2 in range(2):
                kk = g * 2 + b2

                @pl.when(kk < nblk_w)
                def _():
                    pltpu.make_async_copy(
                        feats_hbm.at[pl.ds(0, _R)],
                        rows_v.at[b2, pl.ds(0, _R)], sem).wait()

                    @pl.when(kk + 1 < nblk_w)
                    def _():
                        row_dma(kk + 1, (b2 + 1) % 2)

                    idx = ids_v.at[kk]
                    pltpu.sync_copy(rows_v.at[b2], acc_sh.at[idx], add=True)
                    pltpu.sync_copy(ones_v, cnt_sh.at[idx], add=True)
            return carry

        lax.fori_loop(0, (nblk_w + 1) // 2, pair_body, 0)

        plsc.subcore_barrier()

        @pl.when(s == 0)
        def _():
            pltpu.sync_copy(acc_sh, psum_hbm.at[c])
            pltpu.sync_copy(cnt_sh, pcnt_hbm.at[c])

    return sc_call


def kernel(sfb, node_feats, segment_ids):
    n, d = node_feats.shape
    b = sfb.shape[0]
    nblk = n // _R
    nblk_w = nblk // (_NC * _NS)
    assert nblk * _R == n and nblk_w * _NC * _NS == nblk
    assert d % 16 == 0 and b % 16 == 0

    ids = segment_ids.astype(jnp.int32).reshape(nblk, _R)
    ids_pad = jnp.pad(ids, ((0, 0), (0, _RP - _R)))

    psum, pcnt = _build_sc_call(n, d, b, nblk_w)(node_feats, ids_pad)

    sums = psum[0] + psum[1]
    cnt = pcnt[0, :, 0] + pcnt[1, :, 0]
    g_feat = sums / jnp.maximum(cnt, 1.0)[:, None]
    return jnp.concatenate(
        (sfb, g_feat.reshape(b, d, 1, 1, 1)), axis=1)


# speedup vs baseline: 6.3298x; 1.0912x over previous
import functools

import jax
import jax.numpy as jnp
from jax import lax
from jax.experimental import pallas as pl
from jax.experimental.pallas import tpu as pltpu
from jax.experimental.pallas import tpu_sc as plsc

_NC = 2
_NS = 16
_R = 125
_RP = 128


@functools.lru_cache(maxsize=None)
def _build_sc_call(n, d, b, nblk_w):
    mesh = plsc.VectorSubcoreMesh(core_axis_name="c", subcore_axis_name="s")

    @functools.partial(
        pl.kernel,
        out_type=(
            jax.ShapeDtypeStruct((_NC, b, d), jnp.float32),
            jax.ShapeDtypeStruct((_NC, b, 16), jnp.float32),
        ),
        mesh=mesh,
        compiler_params=pltpu.CompilerParams(use_tc_tiling_on_sc=False),
        scratch_types=[
            pltpu.VMEM((4, _RP, d), jnp.float32),
            pltpu.VMEM((nblk_w, _RP), jnp.int32),
            pltpu.VMEM((_RP, 16), jnp.float32),
            pltpu.VMEM((b, d), jnp.float32),
            pltpu.VMEM((b, 16), jnp.float32),
            pltpu.SemaphoreType.DMA,
            pltpu.VMEM_SHARED((b, d), jnp.float32),
            pltpu.VMEM_SHARED((b, 16), jnp.float32),
        ],
    )
    def sc_call(feats_hbm, ids_hbm, psum_hbm, pcnt_hbm,
                rows_v, ids_v, ones_v, zf_v, zc_v, sem,
                acc_sh, cnt_sh):
        c = lax.axis_index("c")
        s = lax.axis_index("s")
        wid = s * _NC + c

        zeros16 = jnp.zeros((16,), jnp.float32)
        ones16 = jnp.ones((16,), jnp.float32)
        def fill_ones(r, carry):
            ones_v[r, :] = ones16
            return carry

        def fill_ones_pad(r, carry):
            ones_v[r, :] = zeros16
            for bb in range(4):
                for cc in range(d // 16):
                    rows_v[bb, r, pl.ds(cc * 16, 16)] = zeros16
            return carry

        def fill_zf(r, carry):
            for cc in range(d // 16):
                zf_v[r, pl.ds(cc * 16, 16)] = zeros16
            zc_v[r, :] = zeros16
            return carry

        lax.fori_loop(0, _R, fill_ones, 0)
        lax.fori_loop(_R, _RP, fill_ones_pad, 0)
        lax.fori_loop(0, b, fill_zf, 0)

        @pl.when(s == 0)
        def _():
            pltpu.sync_copy(zf_v, acc_sh)
            pltpu.sync_copy(zc_v, cnt_sh)

        plsc.subcore_barrier()

        pltpu.sync_copy(ids_hbm.at[pl.ds(wid * nblk_w, nblk_w)], ids_v)

        def row_dma(kk, buf):
            row0 = (wid * nblk_w + kk) * _R
            return pltpu.async_copy(feats_hbm.at[pl.ds(row0, _R)],
                                    rows_v.at[buf, pl.ds(0, _R)], sem)

        for p in range(3):
            row_dma(p, p)

        def quad_body(g, carry):
            for b2 in range(4):
                kk = g * 4 + b2

                @pl.when(kk < nblk_w)
                def _():
                    pltpu.make_async_copy(
                        feats_hbm.at[pl.ds(0, _R)],
                        rows_v.at[b2, pl.ds(0, _R)], sem).wait()

                    @pl.when(kk + 3 < nblk_w)
                    def _():
                        row_dma(kk + 3, (b2 + 3) % 4)

                    idx = ids_v.at[kk]
                    pltpu.sync_copy(rows_v.at[b2], acc_sh.at[idx], add=True)
                    pltpu.sync_copy(ones_v, cnt_sh.at[idx], add=True)
            return carry

        lax.fori_loop(0, (nblk_w + 3) // 4, quad_body, 0)

        plsc.subcore_barrier()

        @pl.when(s == 0)
        def _():
            pltpu.sync_copy(acc_sh, psum_hbm.at[c])
            pltpu.sync_copy(cnt_sh, pcnt_hbm.at[c])

    return sc_call


def kernel(sfb, node_feats, segment_ids):
    n, d = node_feats.shape
    b = sfb.shape[0]
    nblk = n // _R
    nblk_w = nblk // (_NC * _NS)
    assert nblk * _R == n and nblk_w * _NC * _NS == nblk
    assert d % 16 == 0 and b % 16 == 0

    ids = segment_ids.astype(jnp.int32).reshape(nblk, _R)
    ids_pad = jnp.pad(ids, ((0, 0), (0, _RP - _R)))

    psum, pcnt = _build_sc_call(n, d, b, nblk_w)(node_feats, ids_pad)

    sums = psum[0] + psum[1]
    cnt = pcnt[0, :, 0] + pcnt[1, :, 0]
    g_feat = sums / jnp.maximum(cnt, 1.0)[:, None]
    return jnp.concatenate(
        (sfb, g_feat.reshape(b, d, 1, 1, 1)), axis=1)
